# baseline (speedup 1.0000x reference)
import functools

import jax
import jax.numpy as jnp
from jax import lax
from jax.experimental import pallas as pl
from jax.experimental.pallas import tpu as pltpu
from jax.experimental.pallas import tpu_sc as plsc

N = 1 << 22
ROWS = 2048
COLS = 2048
HALF = 1024
NC, NS = 2, 16
L = 16

TCG = 32
TCB = N // TCG


def _tc_reduce_body(u_ref, psi_ref, stats_ref, acc_ref):
    i = pl.program_id(0)

    @pl.when(i == 0)
    def _():
        acc_ref[0] = 0.0
        acc_ref[1] = 0.0

    x = psi_ref[...].reshape(TCB // COLS, COLS)
    s0 = jnp.sum(x[:, :HALF] * x[:, :HALF])
    s1 = jnp.sum(x[:, HALF:] * x[:, HALF:])
    acc_ref[0] += s0
    acc_ref[1] += s1

    @pl.when(i == TCG - 1)
    def _():
        t0 = acc_ref[0]
        t1 = acc_ref[1]
        total = t0 + t1
        p0 = t0 / total
        outcome = u_ref[0] > p0
        p_out = jnp.where(outcome, 1.0 - p0, p0)
        scale = lax.rsqrt(p_out)
        outf = jnp.where(outcome, 1.0, 0.0)
        iv = lax.iota(jnp.float32, 128)
        iv_i = lax.iota(jnp.int32, 128)
        del iv
        stats_ref[...] = jnp.where(
            iv_i == 0, outf,
            jnp.where(iv_i == 1, p_out,
                      jnp.where(iv_i == 2, scale, 0.0)))


_tc_reduce = pl.pallas_call(
    _tc_reduce_body,
    grid=(TCG,),
    in_specs=[
        pl.BlockSpec(memory_space=pltpu.SMEM),
        pl.BlockSpec((TCB,), lambda i: (i,)),
    ],
    out_specs=pl.BlockSpec((128,), lambda i: (0,)),
    out_shape=jax.ShapeDtypeStruct((128,), jnp.float32),
    scratch_shapes=[pltpu.SMEM((2,), jnp.float32)],
)

RPT2 = ROWS // NS
CH = 16
NCH2 = RPT2 // CH
OE = CH * HALF

_mesh = plsc.VectorSubcoreMesh(core_axis_name="c", subcore_axis_name="s",
                               num_cores=1, num_subcores=NS)


@functools.partial(
    pl.kernel,
    out_type=jax.ShapeDtypeStruct((N // 2,), jnp.float32),
    mesh=_mesh,
    scratch_types=[
        pltpu.VMEM((CH * COLS,), jnp.float32),
        pltpu.VMEM((CH * COLS,), jnp.float32),
        pltpu.VMEM((OE,), jnp.float32),
        pltpu.VMEM((OE,), jnp.float32),
        pltpu.VMEM((L,), jnp.float32),
        pltpu.SemaphoreType.DMA,
        pltpu.SemaphoreType.DMA,
        pltpu.SemaphoreType.DMA,
        pltpu.SemaphoreType.DMA,
    ],
)
def _sc_select(psi_hbm, stats_hbm, out_hbm,
               bufa, bufb, obufa, obufb, st_v, sema, semb, semoa, semob):
    cid = lax.axis_index("c")
    sid = lax.axis_index("s")
    bufs = (bufa, bufb)
    obufs = (obufa, obufb)
    sems = (sema, semb)
    semso = (semoa, semob)

    pltpu.sync_copy(stats_hbm.at[pl.ds(0, L)], st_v)
    st = st_v[...]
    outcome = st[0] > 0.5
    scale = st[2]
    off = jnp.where(outcome, HALF, 0)

    wid = sid
    row2 = wid * RPT2
    obase = wid * RPT2 * HALF

    def start_in(c):
        b = c % 2
        return pltpu.async_copy(
            psi_hbm.at[pl.ds((row2 + c * CH) * COLS, CH * COLS)],
            bufs[b], sems[b])

    def scale_chunk(buf, obuf):
        def body(i, carry):
            r = i >> 4
            q = (i & 15) * 64
            p = r * COLS + off + q
            o = r * HALF + q
            for k in range(4):
                obuf[pl.ds(o + k * L, L)] = buf[pl.ds(p + k * L, L)] * scale
            return carry
        lax.fori_loop(0, CH * 16, body, 0, unroll=4)

    in_copies = [start_in(0), start_in(1)]
    out_copies = [None, None]
    for c in range(NCH2):
        b = c % 2
        in_copies[b].wait()
        if out_copies[b] is not None:
            out_copies[b].wait()
        scale_chunk(bufs[b], obufs[b])
        out_copies[b] = pltpu.async_copy(
            obufs[b], out_hbm.at[pl.ds(obase + c * OE, OE)], semso[b])
        if c + 2 < NCH2:
            in_copies[b] = start_in(c + 2)
    out_copies[0].wait()
    out_copies[1].wait()


def kernel(psi, u):
    u1 = jnp.full((1,), u, jnp.float32)
    stats = _tc_reduce(u1, psi)
    psi_post = _sc_select(psi, stats)
    outcome = stats[0] > 0.5
    p_outcome = stats[1]
    return psi_post, outcome, p_outcome


# speedup vs baseline: 1.4566x; 1.4566x over previous
import functools

import jax
import jax.numpy as jnp
from jax import lax
from jax.experimental import pallas as pl
from jax.experimental.pallas import tpu as pltpu
from jax.experimental.pallas import tpu_sc as plsc

N = 1 << 22
ROWS = 2048
COLS = 2048
HALF = 1024
NC, NS = 2, 16
L = 16

TCG = 16
TCB = N // TCG


def _tc_reduce_body(u_ref, psi_ref, stats_ref, outc_ref, pout_ref, acc_ref):
    i = pl.program_id(0)

    @pl.when(i == 0)
    def _():
        acc_ref[0] = 0.0
        acc_ref[1] = 0.0

    x = psi_ref[...].reshape(TCB // COLS, COLS)
    s0 = jnp.sum(x[:, :HALF] * x[:, :HALF])
    s1 = jnp.sum(x[:, HALF:] * x[:, HALF:])
    acc_ref[0] += s0
    acc_ref[1] += s1

    @pl.when(i == TCG - 1)
    def _():
        t0 = acc_ref[0]
        t1 = acc_ref[1]
        total = t0 + t1
        p0 = t0 / total
        outcome = u_ref[0] > p0
        p_out = jnp.where(outcome, 1.0 - p0, p0)
        scale = lax.rsqrt(p_out)
        outf = jnp.where(outcome, 1.0, 0.0)
        iv_i = lax.iota(jnp.int32, 128)
        stats_ref[...] = jnp.where(
            iv_i == 0, outf,
            jnp.where(iv_i == 1, p_out,
                      jnp.where(iv_i == 2, scale, 0.0)))
        outc_ref[0] = outcome
        pout_ref[0] = p_out


_tc_reduce = pl.pallas_call(
    _tc_reduce_body,
    grid=(TCG,),
    in_specs=[
        pl.BlockSpec(memory_space=pltpu.SMEM),
        pl.BlockSpec((TCB,), lambda i: (i,)),
    ],
    out_specs=(
        pl.BlockSpec((128,), lambda i: (0,)),
        pl.BlockSpec(memory_space=pltpu.SMEM),
        pl.BlockSpec(memory_space=pltpu.SMEM),
    ),
    out_shape=(
        jax.ShapeDtypeStruct((128,), jnp.float32),
        jax.ShapeDtypeStruct((1,), jnp.bool_),
        jax.ShapeDtypeStruct((1,), jnp.float32),
    ),
    scratch_shapes=[pltpu.SMEM((2,), jnp.float32)],
)

RPT2 = ROWS // (NC * NS)
CH = 16
NCH2 = RPT2 // CH
OE = CH * HALF

_mesh = plsc.VectorSubcoreMesh(core_axis_name="c", subcore_axis_name="s",
                               num_cores=NC, num_subcores=NS)


@functools.partial(
    pl.kernel,
    out_type=jax.ShapeDtypeStruct((N // 2,), jnp.float32),
    mesh=_mesh,
    scratch_types=[
        pltpu.VMEM((OE,), jnp.float32),
        pltpu.VMEM((OE,), jnp.float32),
        pltpu.VMEM((OE,), jnp.float32),
        pltpu.VMEM((OE,), jnp.float32),
        pltpu.VMEM((L,), jnp.float32),
        pltpu.SemaphoreType.DMA,
        pltpu.SemaphoreType.DMA,
        pltpu.SemaphoreType.DMA,
        pltpu.SemaphoreType.DMA,
    ],
)
def _sc_select(psi_hbm, stats_hbm, out_hbm,
               bufa, bufb, obufa, obufb, st_v, sema, semb, semoa, semob):
    cid = lax.axis_index("c")
    sid = lax.axis_index("s")
    bufs = (bufa, bufb)
    obufs = (obufa, obufb)
    sems = (sema, semb)
    semso = (semoa, semob)

    pltpu.sync_copy(stats_hbm.at[pl.ds(0, L)], st_v)
    st = st_v[...]
    outcome = st[0] > 0.5
    scale = st[2]
    off = jnp.where(outcome, HALF, 0)

    wid = cid * NS + sid
    row2 = wid * RPT2
    obase = wid * RPT2 * HALF

    def start_in(c):
        b = c % 2
        return [
            pltpu.async_copy(
                psi_hbm.at[pl.ds((row2 + c * CH + r) * COLS + off, HALF)],
                bufs[b].at[pl.ds(r * HALF, HALF)], sems[b])
            for r in range(CH)
        ]

    def scale_chunk(buf, obuf):
        def body(i, carry):
            q = i * 64
            for k in range(4):
                obuf[pl.ds(q + k * L, L)] = buf[pl.ds(q + k * L, L)] * scale
            return carry
        lax.fori_loop(0, CH * 16, body, 0, unroll=4)

    in_copies = [start_in(0), start_in(1)]
    out_copies = [None, None]
    for c in range(NCH2):
        b = c % 2
        for cp in in_copies[b]:
            cp.wait()
        if out_copies[b] is not None:
            out_copies[b].wait()
        scale_chunk(bufs[b], obufs[b])
        out_copies[b] = pltpu.async_copy(
            obufs[b], out_hbm.at[pl.ds(obase + c * OE, OE)], semso[b])
        if c + 2 < NCH2:
            in_copies[b] = start_in(c + 2)
    out_copies[0].wait()
    out_copies[1].wait()


def kernel(psi, u):
    u1 = jnp.full((1,), u, jnp.float32)
    stats, outc, pout = _tc_reduce(u1, psi)
    psi_post = _sc_select(psi, stats)
    return psi_post, outc.reshape(()), pout.reshape(())


# speedup vs baseline: 1.5677x; 1.0763x over previous
import functools

import jax
import jax.numpy as jnp
from jax import lax
from jax.experimental import pallas as pl
from jax.experimental.pallas import tpu as pltpu
from jax.experimental.pallas import tpu_sc as plsc

N = 1 << 22
ROWS = 2048
COLS = 2048
HALF = 1024
NC, NS = 2, 16
L = 16

TCG = 8
TCB = N // TCG


def _tc_reduce_body(u_ref, psi_ref, stats_ref, outc_ref, pout_ref, acc_ref):
    i = pl.program_id(0)

    @pl.when(i == 0)
    def _():
        acc_ref[0] = 0.0
        acc_ref[1] = 0.0

    x = psi_ref[...].reshape(TCB // COLS, COLS)
    s0 = jnp.sum(x[:, :HALF] * x[:, :HALF])
    s1 = jnp.sum(x[:, HALF:] * x[:, HALF:])
    acc_ref[0] += s0
    acc_ref[1] += s1

    @pl.when(i == TCG - 1)
    def _():
        t0 = acc_ref[0]
        t1 = acc_ref[1]
        total = t0 + t1
        p0 = t0 / total
        outcome = u_ref[0] > p0
        p_out = jnp.where(outcome, 1.0 - p0, p0)
        scale = lax.rsqrt(p_out)
        outf = jnp.where(outcome, 1.0, 0.0)
        iv_i = lax.iota(jnp.int32, 128)
        stats_ref[...] = jnp.where(
            iv_i == 0, outf,
            jnp.where(iv_i == 1, p_out,
                      jnp.where(iv_i == 2, scale, 0.0)))
        outc_ref[0] = outcome
        pout_ref[0] = p_out


_tc_reduce = pl.pallas_call(
    _tc_reduce_body,
    grid=(TCG,),
    in_specs=[
        pl.BlockSpec(memory_space=pltpu.SMEM),
        pl.BlockSpec((TCB,), lambda i: (i,)),
    ],
    out_specs=(
        pl.BlockSpec((128,), lambda i: (0,)),
        pl.BlockSpec(memory_space=pltpu.SMEM),
        pl.BlockSpec(memory_space=pltpu.SMEM),
    ),
    out_shape=(
        jax.ShapeDtypeStruct((128,), jnp.float32),
        jax.ShapeDtypeStruct((1,), jnp.bool_),
        jax.ShapeDtypeStruct((1,), jnp.float32),
    ),
    scratch_shapes=[pltpu.SMEM((2,), jnp.float32)],
)

RPT2 = ROWS // (NC * NS)
CH = 16
NCH2 = RPT2 // CH
OE = CH * HALF

_mesh = plsc.VectorSubcoreMesh(core_axis_name="c", subcore_axis_name="s",
                               num_cores=NC, num_subcores=NS)


@functools.partial(
    pl.kernel,
    out_type=jax.ShapeDtypeStruct((N // 2,), jnp.float32),
    mesh=_mesh,
    scratch_types=[
        pltpu.VMEM((OE,), jnp.float32),
        pltpu.VMEM((OE,), jnp.float32),
        pltpu.VMEM((OE,), jnp.float32),
        pltpu.VMEM((OE,), jnp.float32),
        pltpu.VMEM((L,), jnp.float32),
        pltpu.SemaphoreType.DMA,
        pltpu.SemaphoreType.DMA,
        pltpu.SemaphoreType.DMA,
        pltpu.SemaphoreType.DMA,
    ],
)
def _sc_select(psi_hbm, stats_hbm, out_hbm,
               bufa, bufb, obufa, obufb, st_v, sema, semb, semoa, semob):
    cid = lax.axis_index("c")
    sid = lax.axis_index("s")
    bufs = (bufa, bufb)
    obufs = (obufa, obufb)
    sems = (sema, semb)
    semso = (semoa, semob)

    pltpu.sync_copy(stats_hbm.at[pl.ds(0, L)], st_v)
    st = st_v[...]
    outcome = st[0] > 0.5
    scale = st[2]
    off = jnp.where(outcome, HALF, 0)

    wid = cid * NS + sid
    row2 = wid * RPT2
    obase = wid * RPT2 * HALF

    def start_in(c):
        b = c % 2
        return [
            pltpu.async_copy(
                psi_hbm.at[pl.ds((row2 + c * CH + r) * COLS + off, HALF)],
                bufs[b].at[pl.ds(r * HALF, HALF)], sems[b])
            for r in range(CH)
        ]

    def scale_chunk(buf, obuf):
        def body(i, carry):
            q = i * 64
            for k in range(4):
                obuf[pl.ds(q + k * L, L)] = buf[pl.ds(q + k * L, L)] * scale
            return carry
        lax.fori_loop(0, CH * 16, body, 0, unroll=4)

    in_copies = [start_in(0), start_in(1)]
    out_copies = [None, None]
    for c in range(NCH2):
        b = c % 2
        for cp in in_copies[b]:
            cp.wait()
        if out_copies[b] is not None:
            out_copies[b].wait()
        scale_chunk(bufs[b], obufs[b])
        out_copies[b] = pltpu.async_copy(
            obufs[b], out_hbm.at[pl.ds(obase + c * OE, OE)], semso[b])
        if c + 2 < NCH2:
            in_copies[b] = start_in(c + 2)
    out_copies[0].wait()
    out_copies[1].wait()


def kernel(psi, u):
    u1 = u.astype(jnp.float32).reshape(1)
    stats, outc, pout = _tc_reduce(u1, psi)
    psi_post = _sc_select(psi, stats)
    return psi_post, outc.reshape(()), pout.reshape(())
